# baseline (speedup 1.0000x reference)
import functools

import jax
import jax.numpy as jnp
from jax import lax
from jax.experimental import pallas as pl
from jax.experimental.pallas import tpu as pltpu
from jax.experimental.pallas import tpu_sc as plsc

HIDDEN = 1024
EPS = 1e-12

_NUM_CORES = 2
_NUM_SUBCORES = 16
_NW = _NUM_CORES * _NUM_SUBCORES


def _norm_body(t_ref, g_ref, b_ref, o_ref):
    t = t_ref[...]
    mean = jnp.mean(t, axis=-1, keepdims=True)
    cent = t - mean
    var = jnp.mean(cent * cent, axis=-1, keepdims=True)
    o_ref[...] = cent * lax.rsqrt(var + EPS) * g_ref[...] + b_ref[...]


def _normalize_table(table, gamma, beta):
    v, h = table.shape
    return pl.pallas_call(
        _norm_body,
        out_shape=jax.ShapeDtypeStruct((v, h), jnp.float32),
    )(table, gamma.reshape(1, h), beta.reshape(1, h))


@functools.lru_cache(maxsize=None)
def _make_gather(n_tokens, h, chunk):
    bpw = n_tokens // _NW
    nch = bpw // chunk
    mesh = plsc.VectorSubcoreMesh(core_axis_name="c", subcore_axis_name="s")

    @functools.partial(
        pl.kernel,
        mesh=mesh,
        out_type=jax.ShapeDtypeStruct((n_tokens, h), jnp.float32),
        scratch_types=[
            pltpu.VMEM((nch, chunk), jnp.int32),
            pltpu.VMEM((chunk, h), jnp.float32),
            pltpu.VMEM((chunk, h), jnp.float32),
            pltpu.SemaphoreType.DMA,
            pltpu.SemaphoreType.DMA,
            pltpu.SemaphoreType.DMA,
            pltpu.SemaphoreType.DMA,
        ],
    )
    def k(tab_hbm, idx_hbm, out_hbm, idx_v, rows0, rows1, g0, g1, o0, o1):
        wid = lax.axis_index("s") * _NUM_CORES + lax.axis_index("c")
        base = wid * bpw
        pltpu.sync_copy(idx_hbm.at[wid], idx_v)

        rows = (rows0, rows1)
        gsem = (g0, g1)
        osem = (o0, o1)

        def gather(c):
            b = c % 2
            return pltpu.async_copy(tab_hbm.at[idx_v.at[c]], rows[b], gsem[b])

        def put(c):
            b = c % 2
            return pltpu.async_copy(
                rows[b], out_hbm.at[pl.ds(base + c * chunk, chunk)], osem[b]
            )

        g_pending = [gather(0)]
        o_pending = [None, None]
        for c in range(nch):
            b = c % 2
            g_pending.pop(0).wait()
            o_pending[b] = put(c)
            if c + 1 < nch:
                nb = (c + 1) % 2
                if o_pending[nb] is not None:
                    o_pending[nb].wait()
                    o_pending[nb] = None
                g_pending.append(gather(c + 1))
        for p in o_pending:
            if p is not None:
                p.wait()

    return k


def kernel(sentiment_input, table, gamma, beta):
    h = table.shape[1]
    idx = sentiment_input.reshape(-1).astype(jnp.int32)
    n_tokens = idx.shape[0]
    chunk = 32
    ntable = _normalize_table(table, gamma, beta)
    gather = _make_gather(n_tokens, h, chunk)
    out = gather(ntable, idx.reshape(_NW, n_tokens // _NW // chunk, chunk))
    return out.reshape(sentiment_input.shape + (h,))


# speedup vs baseline: 3.3363x; 3.3363x over previous
import functools

import jax
import jax.numpy as jnp
from jax import lax
from jax.experimental import pallas as pl
from jax.experimental.pallas import tpu as pltpu
from jax.experimental.pallas import tpu_sc as plsc

HIDDEN = 1024
EPS = 1e-12

_NUM_CORES = 2
_NUM_SUBCORES = 16
_NW = _NUM_CORES * _NUM_SUBCORES
_LANES = 16


def _norm_body(t_ref, g_ref, b_ref, o_ref):
    t = t_ref[...]
    mean = jnp.mean(t, axis=-1, keepdims=True)
    cent = t - mean
    var = jnp.mean(cent * cent, axis=-1, keepdims=True)
    o_ref[...] = cent * lax.rsqrt(var + EPS) * g_ref[...] + b_ref[...]


def _normalize_table(table, gamma, beta):
    v, h = table.shape
    return pl.pallas_call(
        _norm_body,
        out_shape=jax.ShapeDtypeStruct((v, h), jnp.float32),
    )(table, gamma.reshape(1, h), beta.reshape(1, h))


@functools.lru_cache(maxsize=None)
def _make_scatter(n_tokens, v, h):
    bpw = n_tokens // _NW
    ngrp = bpw // _LANES
    mesh = plsc.VectorSubcoreMesh(core_axis_name="c", subcore_axis_name="s")

    @functools.partial(
        pl.kernel,
        mesh=mesh,
        out_type=jax.ShapeDtypeStruct((n_tokens * h,), jnp.float32),
        scratch_types=[
            pltpu.VMEM((bpw,), jnp.int32),
            pltpu.VMEM((v * h,), jnp.float32),
            pltpu.SemaphoreType.DMA,
        ],
    )
    def k(tab_hbm, idx_hbm, out_hbm, idx_v, tab_v, sem):
        wid = lax.axis_index("s") * _NUM_CORES + lax.axis_index("c")
        base = wid * bpw
        pltpu.sync_copy(tab_hbm, tab_v)
        pltpu.sync_copy(idx_hbm.at[wid], idx_v)

        def group(g, carry):
            goff = pl.multiple_of(g * _LANES, 8)
            idx16 = idx_v[pl.ds(goff, _LANES)]
            gbase = (base + g * _LANES) * h
            for t in range(_LANES):
                src = pl.multiple_of(idx16[t] * h, 8)
                dst = pl.multiple_of(gbase + t * h, 8)
                pltpu.async_copy(
                    tab_v.at[pl.ds(src, h)], out_hbm.at[pl.ds(dst, h)], sem
                )
            return carry

        lax.fori_loop(0, ngrp, group, 0)

        def drain(i, carry):
            pltpu.make_async_copy(
                tab_v.at[pl.ds(0, h)],
                out_hbm.at[pl.ds(base * h, h)],
                sem,
            ).wait()
            return carry

        lax.fori_loop(0, bpw, drain, 0)

    return k


def kernel(sentiment_input, table, gamma, beta):
    v, h = table.shape
    idx = sentiment_input.reshape(-1).astype(jnp.int32)
    n_tokens = idx.shape[0]
    ntable = _normalize_table(table, gamma, beta)
    scatter = _make_scatter(n_tokens, v, h)
    out = scatter(ntable.reshape(-1), idx.reshape(_NW, n_tokens // _NW))
    return out.reshape(sentiment_input.shape + (h,))
